# baseline (speedup 1.0000x reference)
import functools

import jax
import jax.numpy as jnp
from jax import lax
from jax.experimental import pallas as pl
from jax.experimental.pallas import tpu as pltpu
from jax.experimental.pallas import tpu_sc as plsc

_D = 256
_H = 8
_HD = 32
_L = 4
_P = 4
_SPATIAL = ((64, 64), (32, 32), (16, 16), (8, 8))
_V = sum(h * w for h, w in _SPATIAL)
_NC, _NS = 2, 16
_NW = _NC * _NS


def _proj_body(vb, qb, wv, bv, wq, bq, vt_ref, off_ref, aw_ref):
    vt_ref[...] = vb[...] @ wv[...] + bv[...]
    qo = qb[...] @ wq[...] + bq[...]
    off_ref[...] = qo[:, :_D]
    for h in range(_H):
        sl = qo[:, _D + 16 * h:_D + 16 * h + 16]
        m = jnp.max(sl, axis=1, keepdims=True)
        e = jnp.exp(sl - m)
        aw_ref[:, 16 * h:16 * (h + 1)] = e / jnp.sum(e, axis=1, keepdims=True)


def _projections(v2, q2, wv_t, bv, wq_t, bq):
    n = v2.shape[0]
    blk = 256
    grid = n // blk
    return pl.pallas_call(
        _proj_body,
        grid=(grid,),
        in_specs=[
            pl.BlockSpec((blk, _D), lambda i: (i, 0)),
            pl.BlockSpec((blk, _D), lambda i: (i, 0)),
            pl.BlockSpec((_D, _D), lambda i: (0, 0)),
            pl.BlockSpec((1, _D), lambda i: (0, 0)),
            pl.BlockSpec((_D, _D + _H * 16), lambda i: (0, 0)),
            pl.BlockSpec((1, _D + _H * 16), lambda i: (0, 0)),
        ],
        out_specs=[
            pl.BlockSpec((blk, _D), lambda i: (i, 0)),
            pl.BlockSpec((blk, _D), lambda i: (i, 0)),
            pl.BlockSpec((blk, _H * 16), lambda i: (i, 0)),
        ],
        out_shape=[
            jax.ShapeDtypeStruct((n, _D), jnp.float32),
            jax.ShapeDtypeStruct((n, _D), jnp.float32),
            jax.ShapeDtypeStruct((n, _H * 16), jnp.float32),
        ],
    )(v2, q2, wv_t, bv, wq_t, bq)


def _sc_gather_reduce(table, idx2d, w2d):
    nrows = idx2d.shape[0]
    per_tile = nrows // _NW
    chunks = per_tile // 8
    mesh = plsc.VectorSubcoreMesh(core_axis_name="c", subcore_axis_name="s")

    @functools.partial(
        pl.kernel,
        mesh=mesh,
        out_type=jax.ShapeDtypeStruct((nrows * 2, _HD), jnp.float32),
        compiler_params=pltpu.CompilerParams(use_tc_tiling_on_sc=False,
                                             needs_layout_passes=False),
        scratch_types=[
            pltpu.VMEM((8, 128), jnp.int32),
            pltpu.VMEM((8, 128), jnp.int32),
            pltpu.VMEM((8, 128), jnp.float32),
            pltpu.VMEM((8, 128), jnp.float32),
            pltpu.VMEM((1024, _HD), jnp.float32),
            pltpu.VMEM((1024, _HD), jnp.float32),
            pltpu.VMEM((16, _HD), jnp.float32),
            pltpu.SemaphoreType.DMA,
            pltpu.SemaphoreType.DMA,
        ],
    )
    def k(table_hbm, idx_hbm, w_hbm, out_hbm, idx0, idx1, w0, w1, g0, g1,
          out_v, sem0, sem1):
        wid = lax.axis_index("s") * _NC + lax.axis_index("c")
        base = wid * per_tile
        idxb, wb, gb, semb = (idx0, idx1), (w0, w1), (g0, g1), (sem0, sem1)

        def fire(c, b):
            rb = base + c * 8
            pltpu.sync_copy(idx_hbm.at[pl.ds(rb, 8)], idxb[b])
            pltpu.sync_copy(w_hbm.at[pl.ds(rb, 8)], wb[b])
            for j in range(8):
                pltpu.async_copy(table_hbm.at[idxb[b].at[j]],
                                 gb[b].at[pl.ds(j * 128, 128)], semb[b])

        def drain(b):
            for j in range(8):
                pltpu.make_async_copy(table_hbm.at[idxb[b].at[j]],
                                      gb[b].at[pl.ds(j * 128, 128)],
                                      semb[b]).wait()

        def compute_store(c, b):
            g, w = gb[b], wb[b]
            for mloc in range(16):
                def mac(r, acc):
                    row = mloc * 64 + r
                    i0 = jnp.full((16,), row // 128, jnp.int32)
                    i1 = jnp.full((16,), row % 128, jnp.int32)
                    ws = plsc.load_gather(w, [i0, i1])
                    ga = g[row, pl.ds(0, 16)]
                    gc = g[row, pl.ds(16, 16)]
                    return acc[0] + ws * ga, acc[1] + ws * gc
                a0, a1 = lax.fori_loop(
                    0, 64, mac, (jnp.zeros((16,), jnp.float32),
                                 jnp.zeros((16,), jnp.float32)))
                out_v[mloc, pl.ds(0, 16)] = a0
                out_v[mloc, pl.ds(16, 16)] = a1
            pltpu.sync_copy(out_v, out_hbm.at[pl.ds((base + c * 8) * 2, 16)])

        def body(c0, carry):
            fire(c0, 0)
            fire(c0 + 1, 1)
            drain(0)
            compute_store(c0, 0)
            drain(1)
            compute_store(c0 + 1, 1)
            return carry

        lax.fori_loop(0, chunks // 2, lambda i, c: body(i * 2, c), 0)

    return k(table, idx2d, w2d)


def _outproj_body(x_ref, wo, bo, out_ref):
    out_ref[...] = x_ref[...] @ wo[...] + bo[...]


def _outproj(x2, wo_t, bo):
    n = x2.shape[0]
    blk = 256
    return pl.pallas_call(
        _outproj_body,
        grid=(n // blk,),
        in_specs=[
            pl.BlockSpec((blk, _D), lambda i: (i, 0)),
            pl.BlockSpec((_D, _D), lambda i: (0, 0)),
            pl.BlockSpec((1, _D), lambda i: (0, 0)),
        ],
        out_specs=pl.BlockSpec((blk, _D), lambda i: (i, 0)),
        out_shape=jax.ShapeDtypeStruct((n, _D), jnp.float32),
    )(x2, wo_t, bo)


def kernel(query, reference_points, value, spatial_shapes, level_start_index,
           W_off, b_off, W_attn, b_attn, W_v, b_v, W_out, b_out):
    B, Q, _ = query.shape
    V = value.shape[1]
    BQ = B * Q

    q2 = query.reshape(BQ, _D)
    v2 = value.reshape(BQ, _D)
    wq_t = jnp.concatenate([W_off, W_attn], axis=0).T
    bq = jnp.concatenate([b_off, b_attn])[None, :]
    vtab, off, aw = _projections(v2, q2, W_v.T, b_v[None, :], wq_t, bq)

    off6 = off.reshape(BQ, _H, _L, _P, 2)
    aw4 = aw.reshape(BQ, _H, _L, _P)
    rp = reference_points.reshape(BQ, _L, 2)
    wl = jnp.array([w for _, w in _SPATIAL], jnp.float32)
    hl = jnp.array([h for h, _ in _SPATIAL], jnp.float32)
    lsi = jnp.array([0, 4096, 5120, 5376], jnp.int32)
    gx = (rp[:, None, :, None, 0] + off6[..., 0] / wl[None, None, :, None]) * wl[None, None, :, None] - 0.5
    gy = (rp[:, None, :, None, 1] + off6[..., 1] / hl[None, None, :, None]) * hl[None, None, :, None] - 0.5
    x0 = jnp.floor(gx)
    y0 = jnp.floor(gy)
    wx1 = gx - x0
    wx0 = 1.0 - wx1
    wy1 = gy - y0
    wy0 = 1.0 - wy1
    ix = jnp.stack([x0, x0 + 1, x0, x0 + 1], axis=-1)
    iy = jnp.stack([y0, y0, y0 + 1, y0 + 1], axis=-1)
    wc = jnp.stack([wx0 * wy0, wx1 * wy0, wx0 * wy1, wx1 * wy1], axis=-1)
    wlb = wl[None, None, :, None, None]
    hlb = hl[None, None, :, None, None]
    valid = ((ix >= 0) & (ix <= wlb - 1) & (iy >= 0) & (iy <= hlb - 1)).astype(jnp.float32)
    pix = (jnp.clip(iy, 0, hlb - 1) * wlb + jnp.clip(ix, 0, wlb - 1)).astype(jnp.int32) \
        + lsi[None, None, :, None, None]
    b_idx = (jnp.arange(BQ, dtype=jnp.int32) // Q)[:, None, None, None, None]
    h_idx = jnp.arange(_H, dtype=jnp.int32)[None, :, None, None, None]
    row = (b_idx * V + pix) * _H + h_idx
    w_all = (aw4[..., None] * wc * valid).reshape(BQ * 4, 128)

    idx2d = row.reshape(BQ * 4, 128)
    acc = _sc_gather_reduce(vtab.reshape(B * V * _H, _HD), idx2d, w_all)
    out2 = _outproj(acc.reshape(BQ, _D), W_out.T, b_out[None, :])
    return out2.reshape(B, Q, _D)


# speedup vs baseline: 1.4130x; 1.4130x over previous
import functools

import jax
import jax.numpy as jnp
from jax import lax
from jax.experimental import pallas as pl
from jax.experimental.pallas import tpu as pltpu
from jax.experimental.pallas import tpu_sc as plsc

_D = 256
_H = 8
_HD = 32
_L = 4
_P = 4
_SPATIAL = ((64, 64), (32, 32), (16, 16), (8, 8))
_V = sum(h * w for h, w in _SPATIAL)
_NC, _NS = 2, 16
_NW = _NC * _NS


def _proj_body(vb, qb, wv, bv, wq, bq, vt_ref, off_ref, aw_ref):
    vt_ref[...] = vb[...] @ wv[...] + bv[...]
    qo = qb[...] @ wq[...] + bq[...]
    off_ref[...] = qo[:, :_D]
    for h in range(_H):
        sl = qo[:, _D + 16 * h:_D + 16 * h + 16]
        m = jnp.max(sl, axis=1, keepdims=True)
        e = jnp.exp(sl - m)
        aw_ref[:, 16 * h:16 * (h + 1)] = e / jnp.sum(e, axis=1, keepdims=True)


def _projections(v2, q2, wv_t, bv, wq_t, bq):
    n = v2.shape[0]
    blk = 256
    grid = n // blk
    return pl.pallas_call(
        _proj_body,
        grid=(grid,),
        in_specs=[
            pl.BlockSpec((blk, _D), lambda i: (i, 0)),
            pl.BlockSpec((blk, _D), lambda i: (i, 0)),
            pl.BlockSpec((_D, _D), lambda i: (0, 0)),
            pl.BlockSpec((1, _D), lambda i: (0, 0)),
            pl.BlockSpec((_D, _D + _H * 16), lambda i: (0, 0)),
            pl.BlockSpec((1, _D + _H * 16), lambda i: (0, 0)),
        ],
        out_specs=[
            pl.BlockSpec((blk, _D), lambda i: (i, 0)),
            pl.BlockSpec((blk, _D), lambda i: (i, 0)),
            pl.BlockSpec((blk, _H * 16), lambda i: (i, 0)),
        ],
        out_shape=[
            jax.ShapeDtypeStruct((n, _D), jnp.float32),
            jax.ShapeDtypeStruct((n, _D), jnp.float32),
            jax.ShapeDtypeStruct((n, _H * 16), jnp.float32),
        ],
    )(v2, q2, wv_t, bv, wq_t, bq)


def _sc_gather(table, idx2d):
    nrows = idx2d.shape[0]
    per_tile = nrows // _NW
    chunks = per_tile // 8
    mesh = plsc.VectorSubcoreMesh(core_axis_name="c", subcore_axis_name="s")

    @functools.partial(
        pl.kernel,
        mesh=mesh,
        out_type=jax.ShapeDtypeStruct((nrows * 128, _HD), jnp.float32),
        compiler_params=pltpu.CompilerParams(use_tc_tiling_on_sc=False),
        scratch_types=[
            pltpu.VMEM((8, 128), jnp.int32),
            pltpu.VMEM((8, 128), jnp.int32),
            pltpu.VMEM((1024, _HD), jnp.float32),
            pltpu.VMEM((1024, _HD), jnp.float32),
            pltpu.SemaphoreType.DMA,
            pltpu.SemaphoreType.DMA,
        ],
    )
    def k(table_hbm, idx_hbm, out_hbm, idx0, idx1, g0, g1, sem0, sem1):
        wid = lax.axis_index("s") * _NC + lax.axis_index("c")
        base = wid * per_tile
        idxb, gb, semb = (idx0, idx1), (g0, g1), (sem0, sem1)

        def fire(c, b):
            rb = base + c * 8
            pltpu.sync_copy(idx_hbm.at[pl.ds(rb, 8)], idxb[b])
            for j in range(8):
                pltpu.async_copy(table_hbm.at[idxb[b].at[j]],
                                 gb[b].at[pl.ds(j * 128, 128)], semb[b])

        def drain_store(c, b):
            for j in range(8):
                pltpu.make_async_copy(table_hbm.at[idxb[b].at[j]],
                                      gb[b].at[pl.ds(j * 128, 128)],
                                      semb[b]).wait()
            pltpu.sync_copy(gb[b], out_hbm.at[pl.ds((base + c * 8) * 128, 1024)])

        def body(i, carry):
            c0 = i * 2
            fire(c0, 0)
            fire(c0 + 1, 1)
            drain_store(c0, 0)
            drain_store(c0 + 1, 1)
            return carry

        lax.fori_loop(0, chunks // 2, body, 0)

    return k(table, idx2d)


def _reduce_body(g_ref, w_ref, wo, bo, out_ref):
    g = g_ref[...].reshape(16, _H, _L * _P * 4, _HD)
    outs = []
    for h in range(_H):
        wh = w_ref[:, 64 * h:64 * (h + 1)]
        gh = g[:, h]
        outs.append(jnp.sum(gh * wh[:, :, None], axis=1))
    o = jnp.concatenate(outs, axis=1)
    out_ref[...] = o @ wo[...] + bo[...]


def _reduce_out(g, w2, wo_t, bo):
    n = w2.shape[0]
    blk = 16
    grid = n // blk
    return pl.pallas_call(
        _reduce_body,
        grid=(grid,),
        in_specs=[
            pl.BlockSpec((blk * 512, _HD), lambda i: (i, 0)),
            pl.BlockSpec((blk, 512), lambda i: (i, 0)),
            pl.BlockSpec((_D, _D), lambda i: (0, 0)),
            pl.BlockSpec((1, _D), lambda i: (0, 0)),
        ],
        out_specs=pl.BlockSpec((blk, _D), lambda i: (i, 0)),
        out_shape=jax.ShapeDtypeStruct((n, _D), jnp.float32),
    )(g, w2, wo_t, bo)


def kernel(query, reference_points, value, spatial_shapes, level_start_index,
           W_off, b_off, W_attn, b_attn, W_v, b_v, W_out, b_out):
    B, Q, _ = query.shape
    V = value.shape[1]
    BQ = B * Q

    q2 = query.reshape(BQ, _D)
    v2 = value.reshape(BQ, _D)
    wq_t = jnp.concatenate([W_off, W_attn], axis=0).T
    bq = jnp.concatenate([b_off, b_attn])[None, :]
    vtab, off, aw = _projections(v2, q2, W_v.T, b_v[None, :], wq_t, bq)

    off6 = off.reshape(BQ, _H, _L, _P, 2)
    aw4 = aw.reshape(BQ, _H, _L, _P)
    rp = reference_points.reshape(BQ, _L, 2)
    wl = jnp.array([w for _, w in _SPATIAL], jnp.float32)
    hl = jnp.array([h for h, _ in _SPATIAL], jnp.float32)
    lsi = jnp.array([0, 4096, 5120, 5376], jnp.int32)
    gx = (rp[:, None, :, None, 0] + off6[..., 0] / wl[None, None, :, None]) * wl[None, None, :, None] - 0.5
    gy = (rp[:, None, :, None, 1] + off6[..., 1] / hl[None, None, :, None]) * hl[None, None, :, None] - 0.5
    x0 = jnp.floor(gx)
    y0 = jnp.floor(gy)
    wx1 = gx - x0
    wx0 = 1.0 - wx1
    wy1 = gy - y0
    wy0 = 1.0 - wy1
    ix = jnp.stack([x0, x0 + 1, x0, x0 + 1], axis=-1)
    iy = jnp.stack([y0, y0, y0 + 1, y0 + 1], axis=-1)
    wc = jnp.stack([wx0 * wy0, wx1 * wy0, wx0 * wy1, wx1 * wy1], axis=-1)
    wlb = wl[None, None, :, None, None]
    hlb = hl[None, None, :, None, None]
    valid = ((ix >= 0) & (ix <= wlb - 1) & (iy >= 0) & (iy <= hlb - 1)).astype(jnp.float32)
    pix = (jnp.clip(iy, 0, hlb - 1) * wlb + jnp.clip(ix, 0, wlb - 1)).astype(jnp.int32) \
        + lsi[None, None, :, None, None]
    b_idx = (jnp.arange(BQ, dtype=jnp.int32) // Q)[:, None, None, None, None]
    h_idx = jnp.arange(_H, dtype=jnp.int32)[None, :, None, None, None]
    row = (b_idx * V + pix) * _H + h_idx
    w_all = (aw4[..., None] * wc * valid).reshape(BQ, 512)

    idx2d = row.reshape(BQ * 4, 128)
    g = _sc_gather(vtab.reshape(B * V * _H, _HD), idx2d)

    out2 = _reduce_out(g, w_all, W_out.T, b_out[None, :])
    return out2.reshape(B, Q, _D)
